# baseline (speedup 1.0000x reference)
import jax
import jax.numpy as jnp
from jax.experimental import pallas as pl
from jax.experimental.pallas import tpu as pltpu

N_E = 1024
E_DIM = 64
BETA = 0.25
B_TOTAL = 36864
BLK = 6144
GRID = B_TOTAL // BLK


def _vq_body(z_ref, c_ref, zq_ref, idx_ref, loss_ref, perp_ref,
             counts_s, acc_s):
    i = pl.program_id(0)

    @pl.when(i == 0)
    def _init():
        counts_s[...] = jnp.zeros_like(counts_s)
        acc_s[...] = jnp.zeros_like(acc_s)

    c = c_ref[...]
    c_bf = c.astype(jnp.bfloat16)
    cnorm_c = jnp.sum(c * c, axis=1, keepdims=True)
    iota_c = jax.lax.broadcasted_iota(jnp.int32, (N_E, 1), 0).astype(
        jnp.float32)

    zt = z_ref[...].T

    znorm_t = jnp.sum(zt * zt, axis=0, keepdims=True)

    sneg2_t = jax.lax.dot_general(-2.0 * c, zt,
                                  (((1,), (0,)), ((), ())),
                                  preferred_element_type=jnp.float32)
    d_t = (znorm_t + cnorm_c) + sneg2_t

    dmin_t = jnp.min(d_t, axis=0, keepdims=True)
    idxrow_f = jnp.min(jnp.where(d_t == dmin_t, iota_c, float(N_E)),
                       axis=0, keepdims=True)

    one_hot_t = (iota_c == idxrow_f).astype(jnp.bfloat16)
    zq_t = jax.lax.dot_general(c_bf, one_hot_t,
                               (((0,), (0,)), ((), ())),
                               preferred_element_type=jnp.float32)

    zq_ref[...] = (zt + (zq_t - zt)).T
    idx_ref[...] = idxrow_f.astype(jnp.int32).reshape(1, 1, BLK)

    diff = zq_t - zt
    acc_s[...] += jnp.sum(diff * diff, axis=(0, 1), keepdims=True)
    counts_s[...] += jnp.sum(one_hot_t, axis=1, keepdims=True,
                             dtype=jnp.float32)

    @pl.when(i == GRID - 1)
    def _finalize():
        loss_ref[...] = acc_s[...] * ((1.0 + BETA) / (B_TOTAL * E_DIM))
        e_mean = counts_s[...] * (1.0 / B_TOTAL)
        ent = jnp.sum(e_mean * jnp.log(e_mean + 1e-10),
                      axis=(0, 1), keepdims=True)
        perp_ref[...] = jnp.exp(-ent)


def kernel(z, codebook, interpret=False):
    z2 = z.reshape(B_TOTAL, E_DIM)
    zq, idx3, loss, perp = pl.pallas_call(
        _vq_body,
        grid=(GRID,),
        in_specs=[
            pl.BlockSpec((BLK, E_DIM), lambda i: (i, 0)),
            pl.BlockSpec((N_E, E_DIM), lambda i: (0, 0)),
        ],
        out_specs=[
            pl.BlockSpec((BLK, E_DIM), lambda i: (i, 0)),
            pl.BlockSpec((1, 1, BLK), lambda i: (i, 0, 0)),
            pl.BlockSpec((1, 1), lambda i: (0, 0)),
            pl.BlockSpec((1, 1), lambda i: (0, 0)),
        ],
        out_shape=[
            jax.ShapeDtypeStruct((B_TOTAL, E_DIM), jnp.float32),
            jax.ShapeDtypeStruct((GRID, 1, BLK), jnp.int32),
            jax.ShapeDtypeStruct((1, 1), jnp.float32),
            jax.ShapeDtypeStruct((1, 1), jnp.float32),
        ],
        scratch_shapes=[
            pltpu.VMEM((N_E, 1), jnp.float32),
            pltpu.VMEM((1, 1), jnp.float32),
        ],
        interpret=interpret,
    )(z2, codebook)
    idx = idx3.reshape(B_TOTAL)[:, None]
    return (zq, loss.reshape(()), idx, perp.reshape(()))


# speedup vs baseline: 1.1947x; 1.1947x over previous
import jax
import jax.numpy as jnp
from jax.experimental import pallas as pl
from jax.experimental.pallas import tpu as pltpu

N_E = 1024
E_DIM = 64
BETA = 0.25
B_TOTAL = 36864
BLK = 6144
GRID = B_TOTAL // BLK


def _vq_body(zt_ref, c_ref, zq_ref, idx_ref, loss_ref, perp_ref,
             counts_s, acc_s):
    i = pl.program_id(0)

    @pl.when(i == 0)
    def _init():
        counts_s[...] = jnp.zeros_like(counts_s)
        acc_s[...] = jnp.zeros_like(acc_s)

    c = c_ref[...]
    c_bf = c.astype(jnp.bfloat16)
    cnorm_c = jnp.sum(c * c, axis=1, keepdims=True)
    iota_c = jax.lax.broadcasted_iota(jnp.int32, (N_E, 1), 0).astype(
        jnp.float32)

    zt = zt_ref[...]

    znorm_t = jnp.sum(zt * zt, axis=0, keepdims=True)

    sneg2_t = jax.lax.dot_general(-2.0 * c, zt,
                                  (((1,), (0,)), ((), ())),
                                  preferred_element_type=jnp.float32)
    d_t = (znorm_t + cnorm_c) + sneg2_t

    dmin_t = jnp.min(d_t, axis=0, keepdims=True)
    idxrow_f = jnp.min(jnp.where(d_t == dmin_t, iota_c, float(N_E)),
                       axis=0, keepdims=True)

    one_hot_t = (iota_c == idxrow_f).astype(jnp.bfloat16)
    zq_t = jax.lax.dot_general(c_bf, one_hot_t,
                               (((0,), (0,)), ((), ())),
                               preferred_element_type=jnp.float32)

    zq_ref[...] = (zt + (zq_t - zt)).T
    idx_ref[...] = idxrow_f.astype(jnp.int32).reshape(1, 1, BLK)

    diff = zq_t - zt
    acc_s[...] += jnp.sum(diff * diff, axis=(0, 1), keepdims=True)
    counts_s[...] += jnp.sum(one_hot_t, axis=1, keepdims=True,
                             dtype=jnp.float32)

    @pl.when(i == GRID - 1)
    def _finalize():
        loss_ref[...] = acc_s[...] * ((1.0 + BETA) / (B_TOTAL * E_DIM))
        e_mean = counts_s[...] * (1.0 / B_TOTAL)
        ent = jnp.sum(e_mean * jnp.log(e_mean + 1e-10),
                      axis=(0, 1), keepdims=True)
        perp_ref[...] = jnp.exp(-ent)


def kernel(z, codebook, interpret=False):
    z2 = z.reshape(B_TOTAL, E_DIM)
    zt = z2.T
    zq, idx3, loss, perp = pl.pallas_call(
        _vq_body,
        grid=(GRID,),
        in_specs=[
            pl.BlockSpec((E_DIM, BLK), lambda i: (0, i)),
            pl.BlockSpec((N_E, E_DIM), lambda i: (0, 0)),
        ],
        out_specs=[
            pl.BlockSpec((BLK, E_DIM), lambda i: (i, 0)),
            pl.BlockSpec((1, 1, BLK), lambda i: (i, 0, 0)),
            pl.BlockSpec((1, 1), lambda i: (0, 0)),
            pl.BlockSpec((1, 1), lambda i: (0, 0)),
        ],
        out_shape=[
            jax.ShapeDtypeStruct((B_TOTAL, E_DIM), jnp.float32),
            jax.ShapeDtypeStruct((GRID, 1, BLK), jnp.int32),
            jax.ShapeDtypeStruct((1, 1), jnp.float32),
            jax.ShapeDtypeStruct((1, 1), jnp.float32),
        ],
        scratch_shapes=[
            pltpu.VMEM((N_E, 1), jnp.float32),
            pltpu.VMEM((1, 1), jnp.float32),
        ],
        interpret=interpret,
    )(zt, codebook)
    idx = idx3.reshape(B_TOTAL)[:, None]
    return (zq, loss.reshape(()), idx, perp.reshape(()))


# speedup vs baseline: 1.4313x; 1.1980x over previous
import jax
import jax.numpy as jnp
from jax.experimental import pallas as pl
from jax.experimental.pallas import tpu as pltpu

N_E = 1024
E_DIM = 64
BETA = 0.25
B_TOTAL = 36864
BLK = 6144
GRID = B_TOTAL // BLK


def _vq_body(zt_ref, c_ref, zqt_ref, idx_ref, loss_ref, perp_ref,
             counts_s, acc_s):
    i = pl.program_id(0)

    @pl.when(i == 0)
    def _init():
        counts_s[...] = jnp.zeros_like(counts_s)
        acc_s[...] = jnp.zeros_like(acc_s)

    c = c_ref[...]
    c_bf = c.astype(jnp.bfloat16)
    cnorm_c = jnp.sum(c * c, axis=1, keepdims=True)
    iota_c = jax.lax.broadcasted_iota(jnp.int32, (N_E, 1), 0).astype(
        jnp.float32)

    zt = zt_ref[...]

    znorm_t = jnp.sum(zt * zt, axis=0, keepdims=True)

    sneg2_t = jax.lax.dot_general(-2.0 * c, zt,
                                  (((1,), (0,)), ((), ())),
                                  preferred_element_type=jnp.float32)
    d_t = (znorm_t + cnorm_c) + sneg2_t

    dmin_t = jnp.min(d_t, axis=0, keepdims=True)
    idxrow_f = jnp.min(jnp.where(d_t == dmin_t, iota_c, float(N_E)),
                       axis=0, keepdims=True)

    one_hot_t = (iota_c == idxrow_f).astype(jnp.bfloat16)
    zq_t = jax.lax.dot_general(c_bf, one_hot_t,
                               (((0,), (0,)), ((), ())),
                               preferred_element_type=jnp.float32)

    zqt_ref[...] = zt + (zq_t - zt)
    idx_ref[...] = idxrow_f.astype(jnp.int32).reshape(1, 1, BLK)

    diff = zq_t - zt
    acc_s[...] += jnp.sum(diff * diff, axis=(0, 1), keepdims=True)
    counts_s[...] += jnp.sum(one_hot_t, axis=1, keepdims=True,
                             dtype=jnp.float32)

    @pl.when(i == GRID - 1)
    def _finalize():
        loss_ref[...] = acc_s[...] * ((1.0 + BETA) / (B_TOTAL * E_DIM))
        e_mean = counts_s[...] * (1.0 / B_TOTAL)
        ent = jnp.sum(e_mean * jnp.log(e_mean + 1e-10),
                      axis=(0, 1), keepdims=True)
        perp_ref[...] = jnp.exp(-ent)


def kernel(z, codebook, interpret=False):
    z2 = z.reshape(B_TOTAL, E_DIM)
    zt = z2.T
    zqt, idx3, loss, perp = pl.pallas_call(
        _vq_body,
        grid=(GRID,),
        in_specs=[
            pl.BlockSpec((E_DIM, BLK), lambda i: (0, i)),
            pl.BlockSpec((N_E, E_DIM), lambda i: (0, 0)),
        ],
        out_specs=[
            pl.BlockSpec((E_DIM, BLK), lambda i: (0, i)),
            pl.BlockSpec((1, 1, BLK), lambda i: (i, 0, 0)),
            pl.BlockSpec((1, 1), lambda i: (0, 0)),
            pl.BlockSpec((1, 1), lambda i: (0, 0)),
        ],
        out_shape=[
            jax.ShapeDtypeStruct((E_DIM, B_TOTAL), jnp.float32),
            jax.ShapeDtypeStruct((GRID, 1, BLK), jnp.int32),
            jax.ShapeDtypeStruct((1, 1), jnp.float32),
            jax.ShapeDtypeStruct((1, 1), jnp.float32),
        ],
        scratch_shapes=[
            pltpu.VMEM((N_E, 1), jnp.float32),
            pltpu.VMEM((1, 1), jnp.float32),
        ],
        interpret=interpret,
    )(zt, codebook)
    idx = idx3.reshape(B_TOTAL)[:, None]
    return (zqt.T, loss.reshape(()), idx, perp.reshape(()))


# speedup vs baseline: 1.4344x; 1.0021x over previous
import jax
import jax.numpy as jnp
from jax.experimental import pallas as pl
from jax.experimental.pallas import tpu as pltpu

N_E = 1024
E_DIM = 64
BETA = 0.25
B_TOTAL = 36864
BLK = 6144
GRID = B_TOTAL // BLK


def _vq_body(zt_ref, c_ref, zqt_ref, idx_ref, loss_ref, perp_ref,
             counts_s, acc_s):
    i = pl.program_id(0)

    @pl.when(i == 0)
    def _init():
        counts_s[...] = jnp.zeros_like(counts_s)
        acc_s[...] = jnp.zeros_like(acc_s)

    c = c_ref[...]
    c_bf = c.astype(jnp.bfloat16)
    cnorm_c = jnp.sum(c * c, axis=1, keepdims=True)
    iota_c = jax.lax.broadcasted_iota(jnp.int32, (N_E, 1), 0).astype(
        jnp.float32)

    zt = zt_ref[...]

    znorm_t = jnp.sum(zt * zt, axis=0, keepdims=True)

    sneg2_t = jax.lax.dot_general(-2.0 * c, zt,
                                  (((1,), (0,)), ((), ())),
                                  preferred_element_type=jnp.float32)
    d_t = (znorm_t + cnorm_c) + sneg2_t

    dmin_t = jnp.min(d_t, axis=0, keepdims=True)
    idxrow_f = jnp.min(jnp.where(d_t == dmin_t, iota_c, float(N_E)),
                       axis=0, keepdims=True)

    one_hot_t = (iota_c == idxrow_f).astype(jnp.bfloat16)
    zq_t = jax.lax.dot_general(c_bf, one_hot_t,
                               (((0,), (0,)), ((), ())),
                               preferred_element_type=jnp.float32)

    zqt_ref[...] = zt + (zq_t - zt)
    idx_ref[...] = idxrow_f.astype(jnp.int32).reshape(1, 1, BLK)

    diff = zq_t - zt
    acc_s[...] += jnp.sum(diff * diff, axis=(0, 1), keepdims=True)
    counts_s[...] += jnp.sum(one_hot_t, axis=1, keepdims=True,
                             dtype=jnp.float32)

    @pl.when(i == GRID - 1)
    def _finalize():
        loss_ref[...] = acc_s[...] * ((1.0 + BETA) / (B_TOTAL * E_DIM))
        e_mean = counts_s[...] * (1.0 / B_TOTAL)
        ent = jnp.sum(e_mean * jnp.log(e_mean + 1e-10),
                      axis=(0, 1), keepdims=True)
        perp_ref[...] = jnp.exp(-ent)


def kernel(z, codebook):
    z2 = z.reshape(B_TOTAL, E_DIM)
    zt = z2.T
    zqt, idx3, loss, perp = pl.pallas_call(
        _vq_body,
        grid=(GRID,),
        in_specs=[
            pl.BlockSpec((E_DIM, BLK), lambda i: (0, i)),
            pl.BlockSpec((N_E, E_DIM), lambda i: (0, 0)),
        ],
        out_specs=[
            pl.BlockSpec((E_DIM, BLK), lambda i: (0, i)),
            pl.BlockSpec((1, 1, BLK), lambda i: (i, 0, 0)),
            pl.BlockSpec((1, 1), lambda i: (0, 0)),
            pl.BlockSpec((1, 1), lambda i: (0, 0)),
        ],
        out_shape=[
            jax.ShapeDtypeStruct((E_DIM, B_TOTAL), jnp.float32),
            jax.ShapeDtypeStruct((GRID, 1, BLK), jnp.int32),
            jax.ShapeDtypeStruct((1, 1), jnp.float32),
            jax.ShapeDtypeStruct((1, 1), jnp.float32),
        ],
        scratch_shapes=[
            pltpu.VMEM((N_E, 1), jnp.float32),
            pltpu.VMEM((1, 1), jnp.float32),
        ],
    )(zt, codebook)
    idx = idx3.reshape(B_TOTAL)[:, None]
    return (zqt.T, loss.reshape(()), idx, perp.reshape(()))
